# initial kernel scaffold (unmeasured)
import jax
import jax.numpy as jnp
from jax import lax
from jax.experimental import pallas as pl
from jax.experimental.pallas import tpu as pltpu

B = 32
H = 16
D = 128
BS = 32
NP_LOCAL = 256
KL = NP_LOCAL * BS
SCALE = D ** -0.5

_DevIdTy = getattr(pl, "DeviceIdType", None) or pltpu.DeviceIdType
_sem_signal = getattr(pl, "semaphore_signal", None) or pltpu.semaphore_signal
_sem_wait = getattr(pl, "semaphore_wait", None) or pltpu.semaphore_wait
_CompilerParams = getattr(pltpu, "CompilerParams", None) or pltpu.TPUCompilerParams


def _body(w_ref, q_ref, k_ref, v_ref, out_ref,
          o_snd, ml_snd, o_rcv, ml_rcv, send_sems, recv_sems):
    h = pl.program_id(0)
    q = q_ref[0]
    k = k_ref[:, 0, :]
    v = v_ref[:, 0, :]

    s = lax.dot_general(q, k, (((1,), (1,)), ((), ())),
                        preferred_element_type=jnp.float32) * SCALE
    m = jnp.max(s, axis=1, keepdims=True)
    p = jnp.exp(s - m) * w_ref[...]
    l = jnp.sum(p, axis=1, keepdims=True)
    o = lax.dot_general(p, v, (((1,), (0,)), ((), ())),
                        preferred_element_type=jnp.float32)

    o_snd[h] = o
    ml_snd[0, h] = m
    ml_snd[1, h] = l

    @pl.when(h == H - 1)
    def _():
        my_x = lax.axis_index("x")
        my_y = lax.axis_index("y")
        peer = (1 - my_x, my_y)

        barrier = pltpu.get_barrier_semaphore()
        _sem_signal(barrier, inc=1, device_id=peer,
                    device_id_type=_DevIdTy.MESH)
        _sem_wait(barrier, 1)

        rdma_o = pltpu.make_async_remote_copy(
            src_ref=o_snd, dst_ref=o_rcv,
            send_sem=send_sems.at[0], recv_sem=recv_sems.at[0],
            device_id=peer, device_id_type=_DevIdTy.MESH)
        rdma_ml = pltpu.make_async_remote_copy(
            src_ref=ml_snd, dst_ref=ml_rcv,
            send_sem=send_sems.at[1], recv_sem=recv_sems.at[1],
            device_id=peer, device_id_type=_DevIdTy.MESH)
        rdma_o.start()
        rdma_ml.start()
        rdma_o.wait()
        rdma_ml.wait()

        m_s = ml_snd[0]
        l_s = ml_snd[1]
        m_p = ml_rcv[0]
        l_p = ml_rcv[1]
        mm = jnp.maximum(m_s, m_p)
        a = jnp.exp(m_s - mm)
        b = jnp.exp(m_p - mm)
        l_tot = l_s * a + l_p * b
        out_ref[...] = (o_snd[...] * a + o_rcv[...] * b) / l_tot


def kernel(Q, K, V, bt, lens):
    my_x = lax.axis_index("x")
    nb = bt.shape[1]
    valid = jnp.arange(nb)[None, :] < lens[:, None]
    pages = my_x * NP_LOCAL + jnp.arange(NP_LOCAL)
    eq = (bt[:, :, None] == pages[None, None, :]) & valid[:, :, None]
    w_pages = jnp.sum(eq.astype(jnp.float32), axis=1)
    w_tok = jnp.repeat(w_pages, BS, axis=1)

    q_hm = jnp.transpose(Q[:, 0], (1, 0, 2))
    k_flat = K.reshape(KL, H, D)
    v_flat = V.reshape(KL, H, D)

    out_hm = pl.pallas_call(
        _body,
        grid=(H,),
        out_shape=jax.ShapeDtypeStruct((H, B, D), jnp.float32),
        in_specs=[
            pl.BlockSpec((B, KL), lambda h: (0, 0)),
            pl.BlockSpec((1, B, D), lambda h: (h, 0, 0)),
            pl.BlockSpec((KL, 1, D), lambda h: (0, h, 0)),
            pl.BlockSpec((KL, 1, D), lambda h: (0, h, 0)),
        ],
        out_specs=pl.BlockSpec((H, B, D), lambda h: (0, 0, 0)),
        scratch_shapes=[
            pltpu.VMEM((H, B, D), jnp.float32),
            pltpu.VMEM((2, H, B, 1), jnp.float32),
            pltpu.VMEM((H, B, D), jnp.float32),
            pltpu.VMEM((2, H, B, 1), jnp.float32),
            pltpu.SemaphoreType.DMA((2,)),
            pltpu.SemaphoreType.DMA((2,)),
        ],
        compiler_params=_CompilerParams(collective_id=0),
    )(w_tok, q_hm, k_flat, v_flat)

    return jnp.transpose(out_hm, (1, 0, 2))[:, None]


# baseline (device time: 159846 ns/iter reference)
import jax
import jax.numpy as jnp
from jax import lax
from jax.experimental import pallas as pl
from jax.experimental.pallas import tpu as pltpu

B = 32
H = 16
D = 128
BS = 32
NP_LOCAL = 256
KL = NP_LOCAL * BS
SCALE = D ** -0.5

_DevIdTy = getattr(pl, "DeviceIdType", None) or pltpu.DeviceIdType
_sem_signal = getattr(pl, "semaphore_signal", None) or pltpu.semaphore_signal
_sem_wait = getattr(pl, "semaphore_wait", None) or pltpu.semaphore_wait
_CompilerParams = getattr(pltpu, "CompilerParams", None) or pltpu.TPUCompilerParams


def _body(w_ref, q_ref, k_ref, v_ref, out_ref,
          o_snd, ml_snd, o_rcv, ml_rcv, send_sems, recv_sems):
    h = pl.program_id(0)
    q = q_ref[0]
    k = k_ref[...]
    v = v_ref[...]

    s = lax.dot_general(q, k, (((1,), (1,)), ((), ())),
                        preferred_element_type=jnp.float32) * SCALE
    m = jnp.max(s, axis=1, keepdims=True)
    p = jnp.exp(s - m) * w_ref[...]
    l = jnp.sum(p, axis=1, keepdims=True)
    o = lax.dot_general(p, v, (((1,), (0,)), ((), ())),
                        preferred_element_type=jnp.float32)

    o_snd[h] = o
    ml_snd[0, h] = m
    ml_snd[1, h] = l

    @pl.when(h == H - 1)
    def _():
        my_x = lax.axis_index("x")
        my_y = lax.axis_index("y")
        peer = (1 - my_x, my_y)

        barrier = pltpu.get_barrier_semaphore()
        _sem_signal(barrier, inc=1, device_id=peer,
                    device_id_type=_DevIdTy.MESH)
        _sem_wait(barrier, 1)

        rdma_o = pltpu.make_async_remote_copy(
            src_ref=o_snd, dst_ref=o_rcv,
            send_sem=send_sems.at[0], recv_sem=recv_sems.at[0],
            device_id=peer, device_id_type=_DevIdTy.MESH)
        rdma_ml = pltpu.make_async_remote_copy(
            src_ref=ml_snd, dst_ref=ml_rcv,
            send_sem=send_sems.at[1], recv_sem=recv_sems.at[1],
            device_id=peer, device_id_type=_DevIdTy.MESH)
        rdma_o.start()
        rdma_ml.start()
        rdma_o.wait()
        rdma_ml.wait()

        m_s = ml_snd[0]
        l_s = ml_snd[1]
        m_p = ml_rcv[0]
        l_p = ml_rcv[1]
        mm = jnp.maximum(m_s, m_p)
        a = jnp.exp(m_s - mm)
        b = jnp.exp(m_p - mm)
        l_tot = l_s * a + l_p * b
        out_ref[...] = (o_snd[...] * a + o_rcv[...] * b) / l_tot


def kernel(Q, K, V, bt, lens):
    my_x = lax.axis_index("x")
    nb = bt.shape[1]
    valid = jnp.arange(nb)[None, :] < lens[:, None]
    pages = my_x * NP_LOCAL + jnp.arange(NP_LOCAL)
    eq = (bt[:, :, None] == pages[None, None, :]) & valid[:, :, None]
    w_pages = jnp.sum(eq.astype(jnp.float32), axis=1)
    w_tok = jnp.repeat(w_pages, BS, axis=1)

    q_hm = jnp.transpose(Q[:, 0], (1, 0, 2))
    k_flat = K.reshape(KL, H * D)
    v_flat = V.reshape(KL, H * D)

    out_hm = pl.pallas_call(
        _body,
        grid=(H,),
        out_shape=jax.ShapeDtypeStruct((H, B, D), jnp.float32),
        in_specs=[
            pl.BlockSpec((B, KL), lambda h: (0, 0)),
            pl.BlockSpec((1, B, D), lambda h: (h, 0, 0)),
            pl.BlockSpec((KL, D), lambda h: (0, h)),
            pl.BlockSpec((KL, D), lambda h: (0, h)),
        ],
        out_specs=pl.BlockSpec((H, B, D), lambda h: (0, 0, 0)),
        scratch_shapes=[
            pltpu.VMEM((H, B, D), jnp.float32),
            pltpu.VMEM((2, H, B, 1), jnp.float32),
            pltpu.VMEM((H, B, D), jnp.float32),
            pltpu.VMEM((2, H, B, 1), jnp.float32),
            pltpu.SemaphoreType.DMA((2,)),
            pltpu.SemaphoreType.DMA((2,)),
        ],
        compiler_params=_CompilerParams(collective_id=0),
    )(w_tok, q_hm, k_flat, v_flat)

    return jnp.transpose(out_hm, (1, 0, 2))[:, None]


# device time: 149616 ns/iter; 1.0684x vs baseline; 1.0684x over previous
import jax
import jax.numpy as jnp
from jax import lax
from jax.experimental import pallas as pl
from jax.experimental.pallas import tpu as pltpu

B = 32
H = 16
D = 128
BS = 32
NP_LOCAL = 256
KL = NP_LOCAL * BS
NC = 8
C = KL // NC
SCALE = D ** -0.5

_DevIdTy = getattr(pl, "DeviceIdType", None) or pltpu.DeviceIdType
_sem_signal = getattr(pl, "semaphore_signal", None) or pltpu.semaphore_signal
_sem_wait = getattr(pl, "semaphore_wait", None) or pltpu.semaphore_wait
_CompilerParams = getattr(pltpu, "CompilerParams", None) or pltpu.TPUCompilerParams


def _body(w_ref, q_ref, k_ref, v_ref, out_ref,
          o_snd, ml_snd, o_rcv, ml_rcv, send_sems, recv_sems):
    c = pl.program_id(0)

    @pl.when(c == 0)
    def _():
        ml_snd[0] = jnp.full((H, B, 1), -1e30, jnp.float32)
        ml_snd[1] = jnp.zeros((H, B, 1), jnp.float32)

    w = w_ref[...]
    for h in range(H):
        q = q_ref[h]
        k = k_ref[:, h, :]
        v = v_ref[:, h, :]
        s = lax.dot_general(q, k, (((1,), (1,)), ((), ())),
                            preferred_element_type=jnp.float32) * SCALE
        m_c = jnp.max(s, axis=1, keepdims=True)
        m_old = ml_snd[0, h]
        m_new = jnp.maximum(m_old, m_c)
        alpha = jnp.exp(m_old - m_new)
        p = jnp.exp(s - m_new) * w
        l_c = jnp.sum(p, axis=1, keepdims=True)
        pv = lax.dot_general(p, v, (((1,), (0,)), ((), ())),
                             preferred_element_type=jnp.float32)
        ml_snd[0, h] = m_new
        ml_snd[1, h] = ml_snd[1, h] * alpha + l_c

        @pl.when(c == 0)
        def _():
            o_snd[h] = pv

        @pl.when(c != 0)
        def _():
            o_snd[h] = o_snd[h] * alpha + pv

    @pl.when(c == NC - 1)
    def _():
        my_x = lax.axis_index("x")
        my_y = lax.axis_index("y")
        peer = (1 - my_x, my_y)

        barrier = pltpu.get_barrier_semaphore()
        _sem_signal(barrier, inc=1, device_id=peer,
                    device_id_type=_DevIdTy.MESH)
        _sem_wait(barrier, 1)

        rdma_o = pltpu.make_async_remote_copy(
            src_ref=o_snd, dst_ref=o_rcv,
            send_sem=send_sems.at[0], recv_sem=recv_sems.at[0],
            device_id=peer, device_id_type=_DevIdTy.MESH)
        rdma_ml = pltpu.make_async_remote_copy(
            src_ref=ml_snd, dst_ref=ml_rcv,
            send_sem=send_sems.at[1], recv_sem=recv_sems.at[1],
            device_id=peer, device_id_type=_DevIdTy.MESH)
        rdma_o.start()
        rdma_ml.start()
        rdma_o.wait()
        rdma_ml.wait()

        m_s = ml_snd[0]
        l_s = ml_snd[1]
        m_p = ml_rcv[0]
        l_p = ml_rcv[1]
        mm = jnp.maximum(m_s, m_p)
        a = jnp.exp(m_s - mm)
        b = jnp.exp(m_p - mm)
        l_tot = l_s * a + l_p * b
        out_ref[...] = (o_snd[...] * a + o_rcv[...] * b) / l_tot


def kernel(Q, K, V, bt, lens):
    my_x = lax.axis_index("x")
    nb = bt.shape[1]
    valid = jnp.arange(nb)[None, :] < lens[:, None]
    pages = my_x * NP_LOCAL + jnp.arange(NP_LOCAL)
    eq = (bt[:, :, None] == pages[None, None, :]) & valid[:, :, None]
    w_pages = jnp.sum(eq.astype(jnp.float32), axis=1)
    w_tok = jnp.repeat(w_pages, BS, axis=1)

    q_hm = jnp.transpose(Q[:, 0], (1, 0, 2))
    k_flat = K.reshape(KL, H, D)
    v_flat = V.reshape(KL, H, D)

    out_hm = pl.pallas_call(
        _body,
        grid=(NC,),
        out_shape=jax.ShapeDtypeStruct((H, B, D), jnp.float32),
        in_specs=[
            pl.BlockSpec((B, C), lambda c: (0, c)),
            pl.BlockSpec((H, B, D), lambda c: (0, 0, 0)),
            pl.BlockSpec((C, H, D), lambda c: (c, 0, 0)),
            pl.BlockSpec((C, H, D), lambda c: (c, 0, 0)),
        ],
        out_specs=pl.BlockSpec((H, B, D), lambda c: (0, 0, 0)),
        scratch_shapes=[
            pltpu.VMEM((H, B, D), jnp.float32),
            pltpu.VMEM((2, H, B, 1), jnp.float32),
            pltpu.VMEM((H, B, D), jnp.float32),
            pltpu.VMEM((2, H, B, 1), jnp.float32),
            pltpu.SemaphoreType.DMA((2,)),
            pltpu.SemaphoreType.DMA((2,)),
        ],
        compiler_params=_CompilerParams(
            collective_id=0, vmem_limit_bytes=96 * 1024 * 1024),
    )(w_tok, q_hm, k_flat, v_flat)

    return jnp.transpose(out_hm, (1, 0, 2))[:, None]


# device time: 69930 ns/iter; 2.2858x vs baseline; 2.1395x over previous
import jax
import jax.numpy as jnp
from jax import lax
from jax.experimental import pallas as pl
from jax.experimental.pallas import tpu as pltpu

B = 32
H = 16
D = 128
BS = 32
NP_LOCAL = 256
KL = NP_LOCAL * BS
SCALE = D ** -0.5

_DevIdTy = getattr(pl, "DeviceIdType", None) or pltpu.DeviceIdType
_sem_signal = getattr(pl, "semaphore_signal", None) or pltpu.semaphore_signal
_sem_wait = getattr(pl, "semaphore_wait", None) or pltpu.semaphore_wait
_CompilerParams = getattr(pltpu, "CompilerParams", None) or pltpu.TPUCompilerParams
_ANY = getattr(pltpu, "ANY", None) or pl.ANY


def _body(w_ref, q_ref, k_hbm, v_hbm, out_ref,
          k_buf, v_buf, o_snd, ml_snd, o_rcv, ml_rcv,
          k_sems, v_sems, send_sems, recv_sems):
    h = pl.program_id(0)
    slot = lax.rem(h, 2)
    nxt = lax.rem(h + 1, 2)

    @pl.when(h == 0)
    def _():
        pltpu.make_async_copy(k_hbm.at[:, 0, :], k_buf.at[0], k_sems.at[0]).start()
        pltpu.make_async_copy(v_hbm.at[:, 0, :], v_buf.at[0], v_sems.at[0]).start()

    @pl.when(h + 1 < H)
    def _():
        pltpu.make_async_copy(
            k_hbm.at[:, h + 1, :], k_buf.at[nxt], k_sems.at[nxt]).start()
        pltpu.make_async_copy(
            v_hbm.at[:, h + 1, :], v_buf.at[nxt], v_sems.at[nxt]).start()

    pltpu.make_async_copy(k_hbm.at[:, h, :], k_buf.at[slot], k_sems.at[slot]).wait()
    pltpu.make_async_copy(v_hbm.at[:, h, :], v_buf.at[slot], v_sems.at[slot]).wait()

    q = q_ref[h]
    k = k_buf[slot]
    v = v_buf[slot]
    s = lax.dot_general(q, k, (((1,), (1,)), ((), ())),
                        preferred_element_type=jnp.float32) * SCALE
    m = jnp.max(s, axis=1, keepdims=True)
    p = jnp.exp(s - m) * w_ref[...]
    l = jnp.sum(p, axis=1, keepdims=True)
    o = lax.dot_general(p, v, (((1,), (0,)), ((), ())),
                        preferred_element_type=jnp.float32)

    o_snd[h] = o
    ml_snd[0, h] = m
    ml_snd[1, h] = l

    @pl.when(h == H - 1)
    def _():
        my_x = lax.axis_index("x")
        my_y = lax.axis_index("y")
        peer = (1 - my_x, my_y)

        barrier = pltpu.get_barrier_semaphore()
        _sem_signal(barrier, inc=1, device_id=peer,
                    device_id_type=_DevIdTy.MESH)
        _sem_wait(barrier, 1)

        rdma_o = pltpu.make_async_remote_copy(
            src_ref=o_snd, dst_ref=o_rcv,
            send_sem=send_sems.at[0], recv_sem=recv_sems.at[0],
            device_id=peer, device_id_type=_DevIdTy.MESH)
        rdma_ml = pltpu.make_async_remote_copy(
            src_ref=ml_snd, dst_ref=ml_rcv,
            send_sem=send_sems.at[1], recv_sem=recv_sems.at[1],
            device_id=peer, device_id_type=_DevIdTy.MESH)
        rdma_o.start()
        rdma_ml.start()
        rdma_o.wait()
        rdma_ml.wait()

        m_s = ml_snd[0]
        l_s = ml_snd[1]
        m_p = ml_rcv[0]
        l_p = ml_rcv[1]
        mm = jnp.maximum(m_s, m_p)
        a = jnp.exp(m_s - mm)
        b = jnp.exp(m_p - mm)
        l_tot = l_s * a + l_p * b
        out_ref[...] = (o_snd[...] * a + o_rcv[...] * b) / l_tot


def kernel(Q, K, V, bt, lens):
    my_x = lax.axis_index("x")
    nb = bt.shape[1]
    valid = jnp.arange(nb)[None, :] < lens[:, None]
    pages = my_x * NP_LOCAL + jnp.arange(NP_LOCAL)
    eq = (bt[:, :, None] == pages[None, None, :]) & valid[:, :, None]
    w_pages = jnp.sum(eq.astype(jnp.float32), axis=1)
    w_tok = jnp.repeat(w_pages, BS, axis=1)

    q_hm = jnp.transpose(Q[:, 0], (1, 0, 2))
    k_flat = K.reshape(KL, H, D)
    v_flat = V.reshape(KL, H, D)

    out_hm = pl.pallas_call(
        _body,
        grid=(H,),
        out_shape=jax.ShapeDtypeStruct((H, B, D), jnp.float32),
        in_specs=[
            pl.BlockSpec((B, KL), lambda h: (0, 0)),
            pl.BlockSpec((H, B, D), lambda h: (0, 0, 0)),
            pl.BlockSpec(memory_space=_ANY),
            pl.BlockSpec(memory_space=_ANY),
        ],
        out_specs=pl.BlockSpec((H, B, D), lambda h: (0, 0, 0)),
        scratch_shapes=[
            pltpu.VMEM((2, KL, D), jnp.float32),
            pltpu.VMEM((2, KL, D), jnp.float32),
            pltpu.VMEM((H, B, D), jnp.float32),
            pltpu.VMEM((2, H, B, 1), jnp.float32),
            pltpu.VMEM((H, B, D), jnp.float32),
            pltpu.VMEM((2, H, B, 1), jnp.float32),
            pltpu.SemaphoreType.DMA((2,)),
            pltpu.SemaphoreType.DMA((2,)),
            pltpu.SemaphoreType.DMA((2,)),
            pltpu.SemaphoreType.DMA((2,)),
        ],
        compiler_params=_CompilerParams(
            collective_id=0, vmem_limit_bytes=96 * 1024 * 1024),
    )(w_tok, q_hm, k_flat, v_flat)

    return jnp.transpose(out_hm, (1, 0, 2))[:, None]


# device time: 49399 ns/iter; 3.2358x vs baseline; 1.4156x over previous
import jax
import jax.numpy as jnp
from jax import lax
from jax.experimental import pallas as pl
from jax.experimental.pallas import tpu as pltpu

B = 32
H = 16
HG = H // 2
D = 128
BS = 32
NP_LOCAL = 256
KL = NP_LOCAL * BS
SCALE = D ** -0.5

_DevIdTy = getattr(pl, "DeviceIdType", None) or pltpu.DeviceIdType
_sem_signal = getattr(pl, "semaphore_signal", None) or pltpu.semaphore_signal
_sem_wait = getattr(pl, "semaphore_wait", None) or pltpu.semaphore_wait
_CompilerParams = getattr(pltpu, "CompilerParams", None) or pltpu.TPUCompilerParams
_ANY = getattr(pltpu, "ANY", None) or pl.ANY


def _body(w_ref, q_ref, k_hbm, v_hbm, out_ref,
          k_buf, v_buf, o_snd, ml_snd, o_rcv, ml_rcv,
          k_sems, v_sems, send_sems, recv_sems):
    h = pl.program_id(0)
    slot = lax.rem(h, 2)
    nxt = lax.rem(h + 1, 2)
    my_y = lax.axis_index("y")
    oy = my_y * HG

    @pl.when(h == 0)
    def _():
        pltpu.make_async_copy(k_hbm.at[:, oy, :], k_buf.at[0], k_sems.at[0]).start()
        pltpu.make_async_copy(v_hbm.at[:, oy, :], v_buf.at[0], v_sems.at[0]).start()

    @pl.when(h + 1 < HG)
    def _():
        pltpu.make_async_copy(
            k_hbm.at[:, oy + h + 1, :], k_buf.at[nxt], k_sems.at[nxt]).start()
        pltpu.make_async_copy(
            v_hbm.at[:, oy + h + 1, :], v_buf.at[nxt], v_sems.at[nxt]).start()

    pltpu.make_async_copy(k_hbm.at[:, oy + h, :], k_buf.at[slot],
                          k_sems.at[slot]).wait()
    pltpu.make_async_copy(v_hbm.at[:, oy + h, :], v_buf.at[slot],
                          v_sems.at[slot]).wait()

    q = q_ref[oy + h]
    k = k_buf[slot]
    v = v_buf[slot]
    s = lax.dot_general(q, k, (((1,), (1,)), ((), ())),
                        preferred_element_type=jnp.float32) * SCALE
    m = jnp.max(s, axis=1, keepdims=True)
    p = jnp.exp(s - m) * w_ref[...]
    l = jnp.sum(p, axis=1, keepdims=True)
    o = lax.dot_general(p, v, (((1,), (0,)), ((), ())),
                        preferred_element_type=jnp.float32)

    o_snd[h] = o
    ml_snd[0, h] = m
    ml_snd[1, h] = l

    @pl.when(h == HG - 1)
    def _():
        my_x = lax.axis_index("x")
        x_peer = (1 - my_x, my_y)
        y_peer = (my_x, 1 - my_y)

        barrier = pltpu.get_barrier_semaphore()
        for nbr in (x_peer, y_peer):
            _sem_signal(barrier, inc=1, device_id=nbr,
                        device_id_type=_DevIdTy.MESH)
        _sem_wait(barrier, 2)

        rdma_o = pltpu.make_async_remote_copy(
            src_ref=o_snd, dst_ref=o_rcv,
            send_sem=send_sems.at[0], recv_sem=recv_sems.at[0],
            device_id=x_peer, device_id_type=_DevIdTy.MESH)
        rdma_ml = pltpu.make_async_remote_copy(
            src_ref=ml_snd, dst_ref=ml_rcv,
            send_sem=send_sems.at[1], recv_sem=recv_sems.at[1],
            device_id=x_peer, device_id_type=_DevIdTy.MESH)
        rdma_o.start()
        rdma_ml.start()
        rdma_o.wait()
        rdma_ml.wait()

        m_s = ml_snd[0]
        l_s = ml_snd[1]
        m_p = ml_rcv[0]
        l_p = ml_rcv[1]
        mm = jnp.maximum(m_s, m_p)
        a = jnp.exp(m_s - mm)
        b = jnp.exp(m_p - mm)
        l_tot = l_s * a + l_p * b
        out_ref[pl.ds(oy, HG)] = (o_snd[...] * a + o_rcv[...] * b) / l_tot

        rdma_y = pltpu.make_async_remote_copy(
            src_ref=out_ref.at[pl.ds(oy, HG)],
            dst_ref=out_ref.at[pl.ds(oy, HG)],
            send_sem=send_sems.at[2], recv_sem=recv_sems.at[2],
            device_id=y_peer, device_id_type=_DevIdTy.MESH)
        rdma_y.start()
        rdma_y.wait()


def kernel(Q, K, V, bt, lens):
    my_x = lax.axis_index("x")
    nb = bt.shape[1]
    valid = jnp.arange(nb)[None, :] < lens[:, None]
    pages = my_x * NP_LOCAL + jnp.arange(NP_LOCAL)
    eq = (bt[:, :, None] == pages[None, None, :]) & valid[:, :, None]
    w_pages = jnp.sum(eq.astype(jnp.float32), axis=1)
    w_tok = jnp.repeat(w_pages, BS, axis=1)

    q_hm = jnp.transpose(Q[:, 0], (1, 0, 2))
    k_flat = K.reshape(KL, H, D)
    v_flat = V.reshape(KL, H, D)

    out_hm = pl.pallas_call(
        _body,
        grid=(HG,),
        out_shape=jax.ShapeDtypeStruct((H, B, D), jnp.float32),
        in_specs=[
            pl.BlockSpec((B, KL), lambda h: (0, 0)),
            pl.BlockSpec((H, B, D), lambda h: (0, 0, 0)),
            pl.BlockSpec(memory_space=_ANY),
            pl.BlockSpec(memory_space=_ANY),
        ],
        out_specs=pl.BlockSpec((H, B, D), lambda h: (0, 0, 0)),
        scratch_shapes=[
            pltpu.VMEM((2, KL, D), jnp.float32),
            pltpu.VMEM((2, KL, D), jnp.float32),
            pltpu.VMEM((HG, B, D), jnp.float32),
            pltpu.VMEM((2, HG, B, 1), jnp.float32),
            pltpu.VMEM((HG, B, D), jnp.float32),
            pltpu.VMEM((2, HG, B, 1), jnp.float32),
            pltpu.SemaphoreType.DMA((2,)),
            pltpu.SemaphoreType.DMA((2,)),
            pltpu.SemaphoreType.DMA((3,)),
            pltpu.SemaphoreType.DMA((3,)),
        ],
        compiler_params=_CompilerParams(
            collective_id=0, vmem_limit_bytes=96 * 1024 * 1024),
    )(w_tok, q_hm, k_flat, v_flat)

    return jnp.transpose(out_hm, (1, 0, 2))[:, None]


# device time: 45894 ns/iter; 3.4829x vs baseline; 1.0764x over previous
import jax
import jax.numpy as jnp
from jax import lax
from jax.experimental import pallas as pl
from jax.experimental.pallas import tpu as pltpu

B = 32
H = 16
HG = H // 2
D = 128
BS = 32
NP_LOCAL = 256
KL = NP_LOCAL * BS
SCALE = D ** -0.5

_DevIdTy = getattr(pl, "DeviceIdType", None) or pltpu.DeviceIdType
_sem_signal = getattr(pl, "semaphore_signal", None) or pltpu.semaphore_signal
_sem_wait = getattr(pl, "semaphore_wait", None) or pltpu.semaphore_wait
_CompilerParams = getattr(pltpu, "CompilerParams", None) or pltpu.TPUCompilerParams
_ANY = getattr(pltpu, "ANY", None) or pl.ANY


def _x_rdmas(h, x_peer, o_snd, o_rcv, ml_snd, ml_rcv,
             o_ssem, o_rsem, ml_ssem, ml_rsem):
    rdma_o = pltpu.make_async_remote_copy(
        src_ref=o_snd.at[h], dst_ref=o_rcv.at[h],
        send_sem=o_ssem.at[h], recv_sem=o_rsem.at[h],
        device_id=x_peer, device_id_type=_DevIdTy.MESH)
    rdma_ml = pltpu.make_async_remote_copy(
        src_ref=ml_snd.at[h], dst_ref=ml_rcv.at[h],
        send_sem=ml_ssem.at[h], recv_sem=ml_rsem.at[h],
        device_id=x_peer, device_id_type=_DevIdTy.MESH)
    return rdma_o, rdma_ml


def _body(w_ref, q_ref, k_hbm, v_hbm, out_ref,
          k_buf, v_buf, o_snd, ml_snd, o_rcv, ml_rcv,
          k_sems, v_sems, o_ssem, o_rsem, ml_ssem, ml_rsem,
          y_ssem, y_rsem):
    h = pl.program_id(0)
    slot = lax.rem(h, 2)
    nxt = lax.rem(h + 1, 2)
    my_x = lax.axis_index("x")
    my_y = lax.axis_index("y")
    oy = my_y * HG
    x_peer = (1 - my_x, my_y)
    y_peer = (my_x, 1 - my_y)

    @pl.when(h == 0)
    def _():
        barrier = pltpu.get_barrier_semaphore()
        for nbr in (x_peer, y_peer):
            _sem_signal(barrier, inc=1, device_id=nbr,
                        device_id_type=_DevIdTy.MESH)
        _sem_wait(barrier, 2)
        pltpu.make_async_copy(k_hbm.at[:, oy, :], k_buf.at[0], k_sems.at[0]).start()
        pltpu.make_async_copy(v_hbm.at[:, oy, :], v_buf.at[0], v_sems.at[0]).start()

    @pl.when(h + 1 < HG)
    def _():
        pltpu.make_async_copy(
            k_hbm.at[:, oy + h + 1, :], k_buf.at[nxt], k_sems.at[nxt]).start()
        pltpu.make_async_copy(
            v_hbm.at[:, oy + h + 1, :], v_buf.at[nxt], v_sems.at[nxt]).start()

    pltpu.make_async_copy(k_hbm.at[:, oy + h, :], k_buf.at[slot],
                          k_sems.at[slot]).wait()
    pltpu.make_async_copy(v_hbm.at[:, oy + h, :], v_buf.at[slot],
                          v_sems.at[slot]).wait()

    q = q_ref[oy + h]
    k = k_buf[slot]
    v = v_buf[slot]
    s = lax.dot_general(q, k, (((1,), (1,)), ((), ())),
                        preferred_element_type=jnp.float32) * SCALE
    m = jnp.max(s, axis=1, keepdims=True)
    p = jnp.exp(s - m) * w_ref[...]
    l = jnp.sum(p, axis=1, keepdims=True)
    o = lax.dot_general(p, v, (((1,), (0,)), ((), ())),
                        preferred_element_type=jnp.float32)

    o_snd[h] = o
    ml_snd[h, 0] = m
    ml_snd[h, 1] = l

    rdma_o, rdma_ml = _x_rdmas(h, x_peer, o_snd, o_rcv, ml_snd, ml_rcv,
                               o_ssem, o_rsem, ml_ssem, ml_rsem)
    rdma_o.start()
    rdma_ml.start()

    @pl.when(h == HG - 1)
    def _():
        for hh in range(HG):
            ro, rml = _x_rdmas(hh, x_peer, o_snd, o_rcv, ml_snd, ml_rcv,
                               o_ssem, o_rsem, ml_ssem, ml_rsem)
            ro.wait()
            rml.wait()

        m_s = ml_snd[:, 0]
        l_s = ml_snd[:, 1]
        m_p = ml_rcv[:, 0]
        l_p = ml_rcv[:, 1]
        mm = jnp.maximum(m_s, m_p)
        a = jnp.exp(m_s - mm)
        b = jnp.exp(m_p - mm)
        l_tot = l_s * a + l_p * b
        out_ref[pl.ds(oy, HG)] = (o_snd[...] * a + o_rcv[...] * b) / l_tot

        rdma_y = pltpu.make_async_remote_copy(
            src_ref=out_ref.at[pl.ds(oy, HG)],
            dst_ref=out_ref.at[pl.ds(oy, HG)],
            send_sem=y_ssem, recv_sem=y_rsem,
            device_id=y_peer, device_id_type=_DevIdTy.MESH)
        rdma_y.start()
        rdma_y.wait()


def kernel(Q, K, V, bt, lens):
    my_x = lax.axis_index("x")
    nb = bt.shape[1]
    valid = jnp.arange(nb)[None, :] < lens[:, None]
    pages = my_x * NP_LOCAL + jnp.arange(NP_LOCAL)
    eq = (bt[:, :, None] == pages[None, None, :]) & valid[:, :, None]
    w_pages = jnp.sum(eq.astype(jnp.float32), axis=1)
    w_tok = jnp.repeat(w_pages, BS, axis=1)

    q_hm = jnp.transpose(Q[:, 0], (1, 0, 2))
    k_flat = K.reshape(KL, H, D)
    v_flat = V.reshape(KL, H, D)

    out_hm = pl.pallas_call(
        _body,
        grid=(HG,),
        out_shape=jax.ShapeDtypeStruct((H, B, D), jnp.float32),
        in_specs=[
            pl.BlockSpec((B, KL), lambda h: (0, 0)),
            pl.BlockSpec((H, B, D), lambda h: (0, 0, 0)),
            pl.BlockSpec(memory_space=_ANY),
            pl.BlockSpec(memory_space=_ANY),
        ],
        out_specs=pl.BlockSpec((H, B, D), lambda h: (0, 0, 0)),
        scratch_shapes=[
            pltpu.VMEM((2, KL, D), jnp.float32),
            pltpu.VMEM((2, KL, D), jnp.float32),
            pltpu.VMEM((HG, B, D), jnp.float32),
            pltpu.VMEM((HG, 2, B, 1), jnp.float32),
            pltpu.VMEM((HG, B, D), jnp.float32),
            pltpu.VMEM((HG, 2, B, 1), jnp.float32),
            pltpu.SemaphoreType.DMA((2,)),
            pltpu.SemaphoreType.DMA((2,)),
            pltpu.SemaphoreType.DMA((HG,)),
            pltpu.SemaphoreType.DMA((HG,)),
            pltpu.SemaphoreType.DMA((HG,)),
            pltpu.SemaphoreType.DMA((HG,)),
            pltpu.SemaphoreType.DMA,
            pltpu.SemaphoreType.DMA,
        ],
        compiler_params=_CompilerParams(
            collective_id=0, vmem_limit_bytes=96 * 1024 * 1024),
    )(w_tok, q_hm, k_flat, v_flat)

    return jnp.transpose(out_hm, (1, 0, 2))[:, None]


# device time: 45044 ns/iter; 3.5487x vs baseline; 1.0189x over previous
import jax
import jax.numpy as jnp
from jax import lax
from jax.experimental import pallas as pl
from jax.experimental.pallas import tpu as pltpu

B = 32
H = 16
HG = H // 2
D = 128
BS = 32
NP_LOCAL = 256
KL = NP_LOCAL * BS
SCALE = D ** -0.5

_DevIdTy = getattr(pl, "DeviceIdType", None) or pltpu.DeviceIdType
_sem_signal = getattr(pl, "semaphore_signal", None) or pltpu.semaphore_signal
_sem_wait = getattr(pl, "semaphore_wait", None) or pltpu.semaphore_wait
_CompilerParams = getattr(pltpu, "CompilerParams", None) or pltpu.TPUCompilerParams
_ANY = getattr(pltpu, "ANY", None) or pl.ANY


NSPLIT = 1
RS = KL // NSPLIT


def _slab_copies(hbm, head, buf, slot, sems):
    return [
        pltpu.make_async_copy(
            hbm.at[pl.ds(i * RS, RS), head, :],
            buf.at[slot, pl.ds(i * RS, RS), :],
            sems.at[slot, i])
        for i in range(NSPLIT)
    ]


def _x_rdmas(h, x_peer, o_snd, o_rcv, ml_snd, ml_rcv,
             o_ssem, o_rsem, ml_ssem, ml_rsem):
    rdma_o = pltpu.make_async_remote_copy(
        src_ref=o_snd.at[h], dst_ref=o_rcv.at[h],
        send_sem=o_ssem.at[h], recv_sem=o_rsem.at[h],
        device_id=x_peer, device_id_type=_DevIdTy.MESH)
    rdma_ml = pltpu.make_async_remote_copy(
        src_ref=ml_snd.at[h], dst_ref=ml_rcv.at[h],
        send_sem=ml_ssem.at[h], recv_sem=ml_rsem.at[h],
        device_id=x_peer, device_id_type=_DevIdTy.MESH)
    return rdma_o, rdma_ml


def _body(w_ref, q_ref, k_hbm, v_hbm, out_ref,
          k_buf, v_buf, o_snd, ml_snd, o_rcv, ml_rcv,
          k_sems, v_sems, o_ssem, o_rsem, ml_ssem, ml_rsem,
          y_ssem, y_rsem):
    h = pl.program_id(0)
    slot = lax.rem(h, 3)
    my_x = lax.axis_index("x")
    my_y = lax.axis_index("y")
    oy = my_y * HG
    x_peer = (1 - my_x, my_y)
    y_peer = (my_x, 1 - my_y)

    @pl.when(h == 0)
    def _():
        for g in range(2):
            for cp in _slab_copies(k_hbm, oy + g, k_buf, g, k_sems):
                cp.start()
            for cp in _slab_copies(v_hbm, oy + g, v_buf, g, v_sems):
                cp.start()
        barrier = pltpu.get_barrier_semaphore()
        for nbr in (x_peer, y_peer):
            _sem_signal(barrier, inc=1, device_id=nbr,
                        device_id_type=_DevIdTy.MESH)
        _sem_wait(barrier, 2)

    @pl.when(h + 2 < HG)
    def _():
        nxt2 = lax.rem(h + 2, 3)
        for cp in _slab_copies(k_hbm, oy + h + 2, k_buf, nxt2, k_sems):
            cp.start()
        for cp in _slab_copies(v_hbm, oy + h + 2, v_buf, nxt2, v_sems):
            cp.start()

    for cp in _slab_copies(k_hbm, oy + h, k_buf, slot, k_sems):
        cp.wait()
    for cp in _slab_copies(v_hbm, oy + h, v_buf, slot, v_sems):
        cp.wait()

    q = q_ref[oy + h]
    k = k_buf[slot]
    v = v_buf[slot]
    s = lax.dot_general(q, k, (((1,), (1,)), ((), ())),
                        preferred_element_type=jnp.float32) * SCALE
    m = jnp.max(s, axis=1, keepdims=True)
    p = jnp.exp(s - m) * w_ref[...]
    l = jnp.sum(p, axis=1, keepdims=True)
    o = lax.dot_general(p, v, (((1,), (0,)), ((), ())),
                        preferred_element_type=jnp.float32)

    o_snd[h] = o
    ml_snd[h, 0] = m
    ml_snd[h, 1] = l

    rdma_o, rdma_ml = _x_rdmas(h, x_peer, o_snd, o_rcv, ml_snd, ml_rcv,
                               o_ssem, o_rsem, ml_ssem, ml_rsem)
    rdma_o.start()
    rdma_ml.start()

    @pl.when(h == HG - 1)
    def _():
        for hh in range(HG):
            ro, rml = _x_rdmas(hh, x_peer, o_snd, o_rcv, ml_snd, ml_rcv,
                               o_ssem, o_rsem, ml_ssem, ml_rsem)
            ro.wait()
            rml.wait()

        m_s = ml_snd[:, 0]
        l_s = ml_snd[:, 1]
        m_p = ml_rcv[:, 0]
        l_p = ml_rcv[:, 1]
        mm = jnp.maximum(m_s, m_p)
        a = jnp.exp(m_s - mm)
        b = jnp.exp(m_p - mm)
        l_tot = l_s * a + l_p * b
        out_ref[pl.ds(oy, HG)] = (o_snd[...] * a + o_rcv[...] * b) / l_tot

        rdma_y = pltpu.make_async_remote_copy(
            src_ref=out_ref.at[pl.ds(oy, HG)],
            dst_ref=out_ref.at[pl.ds(oy, HG)],
            send_sem=y_ssem.at[0], recv_sem=y_rsem.at[0],
            device_id=y_peer, device_id_type=_DevIdTy.MESH)
        rdma_y.start()
        rdma_y.wait()


def kernel(Q, K, V, bt, lens):
    my_x = lax.axis_index("x")
    nb = bt.shape[1]
    valid = (jnp.arange(nb)[None, :] < lens[:, None]).astype(jnp.float32)
    pages = my_x * NP_LOCAL + jnp.arange(NP_LOCAL)
    onehot = (bt[:, :, None] == pages[None, None, :]).astype(jnp.float32)
    w_pages = jax.lax.dot_general(
        valid[:, None, :], onehot, (((2,), (1,)), ((0,), (0,))),
        preferred_element_type=jnp.float32)[:, 0, :]
    w_tok = jnp.repeat(w_pages, BS, axis=1)

    q_hm = jnp.transpose(Q[:, 0], (1, 0, 2))
    k_flat = K.reshape(KL, H, D)
    v_flat = V.reshape(KL, H, D)

    out_hm = pl.pallas_call(
        _body,
        grid=(HG,),
        out_shape=jax.ShapeDtypeStruct((H, B, D), jnp.float32),
        in_specs=[
            pl.BlockSpec((B, KL), lambda h: (0, 0)),
            pl.BlockSpec((H, B, D), lambda h: (0, 0, 0)),
            pl.BlockSpec(memory_space=_ANY),
            pl.BlockSpec(memory_space=_ANY),
        ],
        out_specs=pl.BlockSpec((H, B, D), lambda h: (0, 0, 0)),
        scratch_shapes=[
            pltpu.VMEM((3, KL, D), jnp.float32),
            pltpu.VMEM((3, KL, D), jnp.float32),
            pltpu.VMEM((HG, B, D), jnp.float32),
            pltpu.VMEM((HG, 2, B, 1), jnp.float32),
            pltpu.VMEM((HG, B, D), jnp.float32),
            pltpu.VMEM((HG, 2, B, 1), jnp.float32),
            pltpu.SemaphoreType.DMA((3, NSPLIT)),
            pltpu.SemaphoreType.DMA((3, NSPLIT)),
            pltpu.SemaphoreType.DMA((HG,)),
            pltpu.SemaphoreType.DMA((HG,)),
            pltpu.SemaphoreType.DMA((HG,)),
            pltpu.SemaphoreType.DMA((HG,)),
            pltpu.SemaphoreType.DMA((HG,)),
            pltpu.SemaphoreType.DMA((HG,)),
        ],
        compiler_params=_CompilerParams(
            collective_id=0, vmem_limit_bytes=96 * 1024 * 1024),
    )(w_tok, q_hm, k_flat, v_flat)

    return jnp.transpose(out_hm, (1, 0, 2))[:, None]
